# baseline (speedup 1.0000x reference)
import jax
import jax.numpy as jnp
from jax.experimental import pallas as pl
from jax.experimental.pallas import tpu as pltpu

_SCORE_THRESH = 0.05
_NMS_THRESH = 0.5
_DETS = 100
_N = 20000
_LANES = 128
_ROWS = 160
_NPAD = _ROWS * _LANES
_NEG_INF = float("-inf")


def _nms_body(x1_ref, y1_ref, x2_ref, y2_ref, s_ref, cls_ref, out_ref,
              bx1_ref, by1_ref, bx2_ref, by2_ref, area_ref, sw_ref):
    x1 = x1_ref[...]
    y1 = y1_ref[...]
    x2 = x2_ref[...]
    y2 = y2_ref[...]
    cls_f = cls_ref[...].astype(jnp.float32)

    max_c = jnp.maximum(jnp.maximum(jnp.max(x1), jnp.max(y1)),
                        jnp.maximum(jnp.max(x2), jnp.max(y2)))
    scale = max_c + 1.0
    offs = cls_f * scale
    bx1_ref[...] = x1 + offs
    by1_ref[...] = y1 + offs
    bx2_ref[...] = x2 + offs
    by2_ref[...] = y2 + offs
    w = jnp.maximum(bx2_ref[...] - bx1_ref[...], 0.0)
    h = jnp.maximum(by2_ref[...] - by1_ref[...], 0.0)
    area_ref[...] = w * h

    idx = (jax.lax.broadcasted_iota(jnp.int32, (_ROWS, _LANES), 0) * _LANES
           + jax.lax.broadcasted_iota(jnp.int32, (_ROWS, _LANES), 1))
    lane = jax.lax.broadcasted_iota(jnp.int32, (1, _LANES), 1)

    s = s_ref[...]
    sw0 = jnp.where(s > _SCORE_THRESH, s, _NEG_INF)
    sw_ref[...] = sw0
    m0 = jnp.max(sw0)
    sel0 = jnp.min(jnp.where(sw0 == m0, idx, _NPAD))

    def step(i, carry):
        m, sel = carry
        selmask = idx == sel

        def pick(a):
            return jnp.max(jnp.where(selmask, a, _NEG_INF))

        ox1 = pick(x1)
        oy1 = pick(y1)
        ox2 = pick(x2)
        oy2 = pick(y2)
        cls_s = pick(cls_f)

        row = jnp.where(lane == 0, ox1,
              jnp.where(lane == 1, oy1,
              jnp.where(lane == 2, ox2,
              jnp.where(lane == 3, oy2,
              jnp.where(lane == 4, m, cls_s)))))
        out_ref[pl.ds(i, 1), :] = row

        offs_s = cls_s * scale
        sx1 = ox1 + offs_s
        sy1 = oy1 + offs_s
        sx2 = ox2 + offs_s
        sy2 = oy2 + offs_s
        area_s = jnp.maximum(sx2 - sx1, 0.0) * jnp.maximum(sy2 - sy1, 0.0)

        xx1 = jnp.maximum(bx1_ref[...], sx1)
        yy1 = jnp.maximum(by1_ref[...], sy1)
        xx2 = jnp.minimum(bx2_ref[...], sx2)
        yy2 = jnp.minimum(by2_ref[...], sy2)
        iw = jnp.maximum(xx2 - xx1, 0.0)
        ih = jnp.maximum(yy2 - yy1, 0.0)
        inter = iw * ih
        iou = inter / (area_ref[...] + area_s - inter + 1e-9)
        supp = (iou > _NMS_THRESH) | selmask
        s_new = jnp.where(supp, _NEG_INF, sw_ref[...])
        sw_ref[...] = s_new

        m_n = jnp.max(s_new)
        sel_n = jnp.min(jnp.where(s_new == m_n, idx, _NPAD))
        return m_n, sel_n

    jax.lax.fori_loop(0, _DETS, step, (m0, sel0))


def _pad2d(v):
    return jnp.pad(v, (0, _NPAD - _N)).reshape(_ROWS, _LANES)


def kernel(boxes, scores, classes):
    x1 = _pad2d(boxes[:, 0])
    y1 = _pad2d(boxes[:, 1])
    x2 = _pad2d(boxes[:, 2])
    y2 = _pad2d(boxes[:, 3])
    s = _pad2d(scores)
    cls = _pad2d(classes)

    out = pl.pallas_call(
        _nms_body,
        out_shape=jax.ShapeDtypeStruct((_DETS, _LANES), jnp.float32),
        scratch_shapes=[pltpu.VMEM((_ROWS, _LANES), jnp.float32)
                        for _ in range(6)],
    )(x1, y1, x2, y2, s, cls)

    kept_boxes = out[:, 0:4]
    kept_scores = out[:, 4]
    kept_classes = out[:, 5].astype(jnp.int32)
    return kept_boxes, kept_scores, kept_classes


# speedup vs baseline: 1.2545x; 1.2545x over previous
import jax
import jax.numpy as jnp
from jax.experimental import pallas as pl
from jax.experimental.pallas import tpu as pltpu

_SCORE_THRESH = 0.05
_NMS_THRESH = 0.5
_DETS = 100
_N = 20000
_LANES = 128
_ROWS = 160
_NPAD = _ROWS * _LANES
_NEG_INF = float("-inf")


def _argmax_first(v, idx):
    m1 = jnp.max(v, axis=0, keepdims=True)
    rcand = jnp.where(v == m1, idx, float(_NPAD))
    r1 = jnp.min(rcand, axis=0, keepdims=True)
    m = jnp.max(m1)
    sel = jnp.min(jnp.where(m1 == m, r1, float(_NPAD)))
    return m, sel


def _nms_body(x1_ref, y1_ref, x2_ref, y2_ref, s_ref, cls_ref, out_ref,
              bx1_ref, by1_ref, bx2_ref, by2_ref, area_ref, sw_ref):
    x1 = x1_ref[...]
    y1 = y1_ref[...]
    x2 = x2_ref[...]
    y2 = y2_ref[...]
    cls_f = cls_ref[...].astype(jnp.float32)

    max_c = jnp.maximum(jnp.maximum(jnp.max(x1), jnp.max(y1)),
                        jnp.maximum(jnp.max(x2), jnp.max(y2)))
    scale = max_c + 1.0
    offs = cls_f * scale
    bx1_ref[...] = x1 + offs
    by1_ref[...] = y1 + offs
    bx2_ref[...] = x2 + offs
    by2_ref[...] = y2 + offs
    w = jnp.maximum(bx2_ref[...] - bx1_ref[...], 0.0)
    h = jnp.maximum(by2_ref[...] - by1_ref[...], 0.0)
    area_ref[...] = w * h

    idx = (jax.lax.broadcasted_iota(jnp.int32, (_ROWS, _LANES), 0) * _LANES
           + jax.lax.broadcasted_iota(jnp.int32, (_ROWS, _LANES), 1)
           ).astype(jnp.float32)
    lane = jax.lax.broadcasted_iota(jnp.int32, (1, _LANES), 1)

    s = s_ref[...]
    sw0 = jnp.where(s > _SCORE_THRESH, s, _NEG_INF)
    sw_ref[...] = sw0
    m0, sel0 = _argmax_first(sw0, idx)

    def step(i, carry):
        m, sel = carry
        sel_i = sel.astype(jnp.int32)
        r = jax.lax.shift_right_logical(sel_i, 7)
        l = jax.lax.bitwise_and(sel_i, _LANES - 1)

        lmask = lane == l

        def pick(ref):
            return jnp.max(jnp.where(lmask, ref[pl.ds(r, 1), :], _NEG_INF))

        ox1 = pick(x1_ref)
        oy1 = pick(y1_ref)
        ox2 = pick(x2_ref)
        oy2 = pick(y2_ref)
        cls_s = jnp.max(jnp.where(lmask,
                                  cls_ref[pl.ds(r, 1), :].astype(jnp.float32),
                                  _NEG_INF))

        row = jnp.where(lane == 0, ox1,
              jnp.where(lane == 1, oy1,
              jnp.where(lane == 2, ox2,
              jnp.where(lane == 3, oy2,
              jnp.where(lane == 4, m, cls_s)))))
        out_ref[pl.ds(i, 1), :] = row

        offs_s = cls_s * scale
        sx1 = ox1 + offs_s
        sy1 = oy1 + offs_s
        sx2 = ox2 + offs_s
        sy2 = oy2 + offs_s
        area_s = jnp.maximum(sx2 - sx1, 0.0) * jnp.maximum(sy2 - sy1, 0.0)

        xx1 = jnp.maximum(bx1_ref[...], sx1)
        yy1 = jnp.maximum(by1_ref[...], sy1)
        xx2 = jnp.minimum(bx2_ref[...], sx2)
        yy2 = jnp.minimum(by2_ref[...], sy2)
        iw = jnp.maximum(xx2 - xx1, 0.0)
        ih = jnp.maximum(yy2 - yy1, 0.0)
        inter = iw * ih
        iou = inter / (area_ref[...] + area_s - inter + 1e-9)
        supp = (iou > _NMS_THRESH) | (idx == sel)
        s_new = jnp.where(supp, _NEG_INF, sw_ref[...])
        sw_ref[...] = s_new

        return _argmax_first(s_new, idx)

    jax.lax.fori_loop(0, _DETS, step, (m0, sel0))


def _pad2d(v):
    return jnp.pad(v, (0, _NPAD - _N)).reshape(_ROWS, _LANES)


def kernel(boxes, scores, classes):
    x1 = _pad2d(boxes[:, 0])
    y1 = _pad2d(boxes[:, 1])
    x2 = _pad2d(boxes[:, 2])
    y2 = _pad2d(boxes[:, 3])
    s = _pad2d(scores)
    cls = _pad2d(classes)

    out = pl.pallas_call(
        _nms_body,
        out_shape=jax.ShapeDtypeStruct((_DETS, _LANES), jnp.float32),
        scratch_shapes=[pltpu.VMEM((_ROWS, _LANES), jnp.float32)
                        for _ in range(6)],
    )(x1, y1, x2, y2, s, cls)

    kept_boxes = out[:, 0:4]
    kept_scores = out[:, 4]
    kept_classes = out[:, 5].astype(jnp.int32)
    return kept_boxes, kept_scores, kept_classes


# speedup vs baseline: 1.3737x; 1.0951x over previous
import jax
import jax.numpy as jnp
from jax.experimental import pallas as pl
from jax.experimental.pallas import tpu as pltpu

_SCORE_THRESH = 0.05
_NMS_THRESH = 0.5
_DETS = 100
_N = 20000
_LANES = 128
_ROWS = 160
_NPAD = _ROWS * _LANES
_NEG_INF = float("-inf")


def _argmax_first(v, idx):
    m1 = jnp.max(v, axis=0, keepdims=True)
    rcand = jnp.where(v == m1, idx, float(_NPAD))
    r1 = jnp.min(rcand, axis=0, keepdims=True)
    m = jnp.max(m1)
    sel = jnp.min(jnp.where(m1 == m, r1, float(_NPAD)))
    return m, sel


def _nms_body(x1_ref, y1_ref, x2_ref, y2_ref, s_ref, cls_ref,
              x1s_ref, y1s_ref, x2s_ref, y2s_ref, clss_ref, out_ref,
              bx1_ref, by1_ref, bx2_ref, by2_ref, area_ref, sw_ref):
    x1 = x1_ref[...]
    y1 = y1_ref[...]
    x2 = x2_ref[...]
    y2 = y2_ref[...]
    cls_f = cls_ref[...].astype(jnp.float32)

    max_c = jnp.maximum(jnp.maximum(jnp.max(x1), jnp.max(y1)),
                        jnp.maximum(jnp.max(x2), jnp.max(y2)))
    scale = max_c + 1.0
    offs = cls_f * scale
    bx1_ref[...] = x1 + offs
    by1_ref[...] = y1 + offs
    bx2_ref[...] = x2 + offs
    by2_ref[...] = y2 + offs
    w = jnp.maximum(bx2_ref[...] - bx1_ref[...], 0.0)
    h = jnp.maximum(by2_ref[...] - by1_ref[...], 0.0)
    area_ref[...] = w * h

    idx = (jax.lax.broadcasted_iota(jnp.int32, (_ROWS, _LANES), 0) * _LANES
           + jax.lax.broadcasted_iota(jnp.int32, (_ROWS, _LANES), 1)
           ).astype(jnp.float32)
    lane = jax.lax.broadcasted_iota(jnp.int32, (1, _LANES), 1)

    s = s_ref[...]
    sw0 = jnp.where(s > _SCORE_THRESH, s, _NEG_INF)
    sw_ref[...] = sw0
    m0, sel0 = _argmax_first(sw0, idx)

    def step(i, carry):
        m, sel = carry
        sel_i = sel.astype(jnp.int32)

        ox1 = x1s_ref[sel_i]
        oy1 = y1s_ref[sel_i]
        ox2 = x2s_ref[sel_i]
        oy2 = y2s_ref[sel_i]
        cls_s = clss_ref[sel_i]

        row = jnp.where(lane == 0, ox1,
              jnp.where(lane == 1, oy1,
              jnp.where(lane == 2, ox2,
              jnp.where(lane == 3, oy2,
              jnp.where(lane == 4, m, cls_s)))))
        out_ref[pl.ds(i, 1), :] = row

        offs_s = cls_s * scale
        sx1 = ox1 + offs_s
        sy1 = oy1 + offs_s
        sx2 = ox2 + offs_s
        sy2 = oy2 + offs_s
        area_s = jnp.maximum(sx2 - sx1, 0.0) * jnp.maximum(sy2 - sy1, 0.0)

        xx1 = jnp.maximum(bx1_ref[...], sx1)
        yy1 = jnp.maximum(by1_ref[...], sy1)
        xx2 = jnp.minimum(bx2_ref[...], sx2)
        yy2 = jnp.minimum(by2_ref[...], sy2)
        iw = jnp.maximum(xx2 - xx1, 0.0)
        ih = jnp.maximum(yy2 - yy1, 0.0)
        inter = iw * ih
        iou = inter / (area_ref[...] + area_s - inter + 1e-9)
        supp = (iou > _NMS_THRESH) | (idx == sel)
        s_new = jnp.where(supp, _NEG_INF, sw_ref[...])
        sw_ref[...] = s_new

        return _argmax_first(s_new, idx)

    jax.lax.fori_loop(0, _DETS, step, (m0, sel0), unroll=2)


def _pad2d(v):
    return jnp.pad(v, (0, _NPAD - _N)).reshape(_ROWS, _LANES)


def _pad1d(v):
    return jnp.pad(v, (0, _NPAD - _N))


def kernel(boxes, scores, classes):
    x1 = _pad2d(boxes[:, 0])
    y1 = _pad2d(boxes[:, 1])
    x2 = _pad2d(boxes[:, 2])
    y2 = _pad2d(boxes[:, 3])
    s = _pad2d(scores)
    cls = _pad2d(classes)
    x1s = _pad1d(boxes[:, 0])
    y1s = _pad1d(boxes[:, 1])
    x2s = _pad1d(boxes[:, 2])
    y2s = _pad1d(boxes[:, 3])
    clss = _pad1d(classes.astype(jnp.float32))

    vspec = pl.BlockSpec(memory_space=pltpu.VMEM)
    sspec = pl.BlockSpec(memory_space=pltpu.SMEM)
    out = pl.pallas_call(
        _nms_body,
        out_shape=jax.ShapeDtypeStruct((_DETS, _LANES), jnp.float32),
        in_specs=[vspec] * 6 + [sspec] * 5,
        scratch_shapes=[pltpu.VMEM((_ROWS, _LANES), jnp.float32)
                        for _ in range(6)],
    )(x1, y1, x2, y2, s, cls, x1s, y1s, x2s, y2s, clss)

    kept_boxes = out[:, 0:4]
    kept_scores = out[:, 4]
    kept_classes = out[:, 5].astype(jnp.int32)
    return kept_boxes, kept_scores, kept_classes
